# baseline (speedup 1.0000x reference)
import functools

import jax
import jax.numpy as jnp
from jax import lax
from jax.experimental import pallas as pl
from jax.experimental.pallas import tpu as pltpu
from jax.experimental.pallas import tpu_sc as plsc

INT_MIN = -(2**31)
INT_MAX = 2**31 - 1

N_IMG = 16
HW = 512 * 512
N_RES = 8 * HW
N_KEEP = int(N_RES * 0.8)
LN = 128
SL = HW // LN
CSL = 256
NCH = SL // CSL


def _key_u(x):
    i = lax.bitcast_convert_type(x, jnp.int32)
    return jnp.where(i >= 0, i ^ jnp.int32(INT_MIN), ~i)


def _inv_key_u(u):
    i = jnp.where(u < 0, u ^ jnp.int32(INT_MIN), ~u)
    return lax.bitcast_convert_type(i, jnp.float32)


def _medres_body(p_ref, t_ref, r_ref, u_ref):
    def prologue(c, _):
        sl = pl.ds(c * CSL, CSL)
        u_ref[0:8, sl, :] = _key_u(p_ref[:, sl, :])
        u_ref[8:16, sl, :] = _key_u(t_ref[:, sl, :])
        return 0

    lax.fori_loop(0, NCH, prologue, 0)

    def bit_body(it, carry):
        prefix, rr = carry
        b = 31 - it
        maskhi = lax.shift_left(jnp.int32(-1), b)
        bit = lax.shift_left(jnp.int32(1), b)
        pref3 = prefix[:, :, None]

        def cbody(c, acc):
            u = u_ref[:, pl.ds(c * CSL, CSL), :]
            return acc + jnp.sum(((u & maskhi) == pref3).astype(jnp.int32),
                                 axis=1)

        acc = lax.fori_loop(0, NCH, cbody,
                            jnp.zeros((N_IMG, LN), jnp.int32))
        cnt0 = jnp.sum(acc, axis=1, keepdims=True)
        take1 = rr >= cnt0
        prefix = jnp.where(take1, prefix | bit, prefix)
        rr = jnp.where(take1, rr - cnt0, rr)
        return prefix, rr

    rank = jnp.full((N_IMG, 1), HW // 2 - 1, jnp.int32)
    u1, _ = lax.fori_loop(0, 32, bit_body, (jnp.zeros_like(rank), rank))
    s1 = (u1 ^ jnp.int32(INT_MIN))[:, :, None]

    def cbody2(c, carry):
        c_le, amin = carry
        s = u_ref[:, pl.ds(c * CSL, CSL), :] ^ jnp.int32(INT_MIN)
        c_le = c_le + jnp.sum((s <= s1).astype(jnp.int32), axis=1)
        amin = jnp.minimum(amin, jnp.min(
            jnp.where(s > s1, s, jnp.int32(INT_MAX)), axis=1))
        return c_le, amin

    c_le_l, amin_l = lax.fori_loop(
        0, NCH, cbody2,
        (jnp.zeros((N_IMG, LN), jnp.int32),
         jnp.full((N_IMG, LN), INT_MAX, jnp.int32)))
    c_le = jnp.sum(c_le_l, axis=1, keepdims=True)
    above_min = jnp.min(amin_l, axis=1, keepdims=True)
    u2 = jnp.where(c_le >= HW // 2 + 1, u1, above_min ^ jnp.int32(INT_MIN))
    med = 0.5 * (_inv_key_u(u1) + _inv_key_u(u2))
    med3 = med[:, :, None]

    def cbody3(c, acc):
        x = _inv_key_u(u_ref[:, pl.ds(c * CSL, CSL), :])
        return acc + jnp.sum(jnp.abs(x - med3), axis=1)

    sabs_l = lax.fori_loop(0, NCH, cbody3, jnp.zeros((N_IMG, LN), jnp.float32))
    sabs = jnp.sum(sabs_l, axis=1, keepdims=True)
    scale = jnp.maximum(sabs * (1.0 / HW), 1e-6)
    inv_scale = 1.0 / scale
    mp = med3[0:8]
    mt = med3[8:16]
    isp = inv_scale[0:8, :, None]
    ist = inv_scale[8:16, :, None]

    def cbody4(c, _):
        sl = pl.ds(c * CSL, CSL)
        r_ref[:, sl, :] = jnp.abs((p_ref[:, sl, :] - mp) * isp
                                  - (t_ref[:, sl, :] - mt) * ist)
        return 0

    lax.fori_loop(0, NCH, cbody4, 0)


_SC_MESH = plsc.VectorSubcoreMesh(core_axis_name="c", subcore_axis_name="s")

SC_TILES = 16
SC_EPT = N_RES // SC_TILES
SC_CHUNK = 8192
SC_NCHUNK = SC_EPT // SC_CHUNK
SC_VPC = SC_CHUNK // 16


def _sc_lane_reduce(ref1d, j, stride):
    acc = jnp.zeros((16,), jnp.int32)
    for l in range(16):
        acc = acc + ref1d[pl.ds(l * stride + j * 16, 16)]
    return acc


@functools.partial(
    pl.kernel,
    mesh=_SC_MESH,
    compiler_params=pltpu.CompilerParams(needs_layout_passes=False),
    out_type=(
        jax.ShapeDtypeStruct((2 * SC_TILES, 16), jnp.float32),
        jax.ShapeDtypeStruct((16,), jnp.int32),
    ),
    scratch_types=[
        pltpu.VMEM((16 * 2048,), jnp.int32),
        pltpu.VMEM((SC_CHUNK,), jnp.float32),
        pltpu.VMEM((2048,), jnp.int32),
        pltpu.VMEM((16 * 2048,), jnp.int32),
        pltpu.VMEM((16,), jnp.float32),
        pltpu.VMEM((16,), jnp.int32),
        pltpu.VMEM_SHARED((16 * 2048,), jnp.int32),
    ],
)
def _sc_trim(r_hbm, zeros_hbm, outf_hbm, outb_hbm,
             hist, data, mytot, tot, stage_f, stage_b, sh_tot):
    c = lax.axis_index("c")
    s = lax.axis_index("s")

    @pl.when(c == 0)
    def _():
        li = lax.iota(jnp.int32, 16)
        ones = jnp.ones((16,), jnp.int32)
        base = s * SC_EPT

        def hist_pass(shift, nbits, prefix, pshift, rrem):
            pltpu.sync_copy(zeros_hbm, hist)

            def chunk_body(ci, _):
                pltpu.sync_copy(r_hbm.at[pl.ds(base + ci * SC_CHUNK, SC_CHUNK)],
                                data)

                def vec_body(j, _):
                    k = lax.bitcast_convert_type(data[pl.ds(j * 16, 16)],
                                                 jnp.int32)
                    bins = lax.shift_right_logical(k, shift) & ((1 << nbits) - 1)
                    m = lax.shift_right_logical(k, pshift) == prefix
                    plsc.addupdate_scatter(hist, [li * 2048 + bins], ones,
                                           mask=m)
                    return 0

                lax.fori_loop(0, SC_VPC, vec_body, 0)
                return 0

            lax.fori_loop(0, SC_NCHUNK, chunk_body, 0)

            def red_body(j, _):
                mytot[pl.ds(j * 16, 16)] = _sc_lane_reduce(hist, j, 2048)
                return 0

            lax.fori_loop(0, (1 << nbits) // 16, red_body, 0)
            plsc.subcore_barrier()
            pltpu.sync_copy(mytot.at[pl.ds(0, 1 << nbits)],
                            sh_tot.at[pl.ds(s * 2048, 1 << nbits)])
            plsc.subcore_barrier()
            pltpu.sync_copy(sh_tot, tot)

            def scan_body(j, carry):
                found, dig, run, rr = carry
                t = _sc_lane_reduce(tot, j, 2048)
                cum = plsc.cumsum(t)
                tot_j = jnp.sum(t)
                fmask = (run + cum) > rr
                lane = jnp.sum(jnp.where(fmask, 0, 1))
                anyf = lane < 16
                c_at = jnp.sum(jnp.where(li == lane, cum, 0))
                t_at = jnp.sum(jnp.where(li == lane, t, 0))
                newly = jnp.logical_and(found == 0, anyf)
                dig = jnp.where(newly, j * 16 + lane, dig)
                rr = jnp.where(newly, rr - (run + c_at - t_at), rr)
                found = jnp.where(newly, 1, found)
                run = run + tot_j
                return found, dig, run, rr

            nj = (1 << nbits) // 16
            _, dig, _, rrem = lax.fori_loop(
                0, nj, scan_body,
                (jnp.int32(0), jnp.int32(0), jnp.int32(0), rrem))
            return dig, rrem

        rr0 = jnp.int32(N_KEEP - 1)
        b1, rr1 = hist_pass(20, 11, jnp.int32(0), 31, rr0)
        b2, rr2 = hist_pass(9, 11, b1, 20, rr1)
        p12 = lax.shift_left(b1, 11) | b2
        b3, _ = hist_pass(0, 9, p12, 9, rr2)
        t_bits = lax.shift_left(p12, 9) | b3

        def sum_chunk(ci, carry):
            cacc, sacc = carry
            pltpu.sync_copy(r_hbm.at[pl.ds(base + ci * SC_CHUNK, SC_CHUNK)],
                            data)

            def vec_body(j, cc):
                ca, sa = cc
                v = data[pl.ds(j * 16, 16)]
                k = lax.bitcast_convert_type(v, jnp.int32)
                m = k < t_bits
                ca = ca + jnp.where(m, 1.0, 0.0)
                sa = sa + jnp.where(m, v, 0.0)
                return ca, sa

            return lax.fori_loop(0, SC_VPC, vec_body, (cacc, sacc))

        cacc, sacc = lax.fori_loop(
            0, SC_NCHUNK, sum_chunk,
            (jnp.zeros((16,), jnp.float32), jnp.zeros((16,), jnp.float32)))
        stage_f[...] = cacc
        pltpu.sync_copy(stage_f, outf_hbm.at[s])
        stage_f[...] = sacc
        pltpu.sync_copy(stage_f, outf_hbm.at[s + 16])

        @pl.when(s == 0)
        def _():
            stage_b[...] = jnp.full((16,), 1, jnp.int32) * t_bits
            pltpu.sync_copy(stage_b, outb_hbm)


@jax.jit
def kernel(pred_depth, target, mask):
    del mask
    p = pred_depth.reshape(8, SL, LN)
    t = target.reshape(8, SL, LN)
    r = pl.pallas_call(
        _medres_body,
        out_shape=jax.ShapeDtypeStruct((8, SL, LN), jnp.float32),
        scratch_shapes=[pltpu.VMEM((N_IMG, SL, LN), jnp.int32)],
    )(p, t)
    outf, outb = _sc_trim(r.reshape(N_RES),
                          jnp.zeros((16 * 2048,), jnp.int32))
    c_lt = jnp.sum(outf[0:16])
    s_lt = jnp.sum(outf[16:32])
    t_val = lax.bitcast_convert_type(outb[0], jnp.float32)
    total = s_lt + (N_KEEP - c_lt) * t_val
    return total * (1.0 / N_RES)


# speedup vs baseline: 1.0761x; 1.0761x over previous
import functools

import jax
import jax.numpy as jnp
from jax import lax
from jax.experimental import pallas as pl
from jax.experimental.pallas import tpu as pltpu
from jax.experimental.pallas import tpu_sc as plsc

INT_MIN = -(2**31)
INT_MAX = 2**31 - 1

N_IMG = 16
HW = 512 * 512
N_RES = 8 * HW
N_KEEP = int(N_RES * 0.8)
LN = 128
SL = HW // LN
CSL = 256
NCH = SL // CSL


def _key_u(x):
    i = lax.bitcast_convert_type(x, jnp.int32)
    return jnp.where(i >= 0, i ^ jnp.int32(INT_MIN), ~i)


def _inv_key_u(u):
    i = jnp.where(u < 0, u ^ jnp.int32(INT_MIN), ~u)
    return lax.bitcast_convert_type(i, jnp.float32)


def _medres_body(p_ref, t_ref, r_ref, u_ref):
    def prologue(c, _):
        sl = pl.ds(c * CSL, CSL)
        u_ref[0:8, sl, :] = _key_u(p_ref[:, sl, :])
        u_ref[8:16, sl, :] = _key_u(t_ref[:, sl, :])
        return 0

    lax.fori_loop(0, NCH, prologue, 0)

    def bit_body(it, carry):
        prefix, rr = carry
        b = 31 - it
        maskhi = lax.shift_left(jnp.int32(-1), b)
        bit = lax.shift_left(jnp.int32(1), b)
        pref3 = prefix[:, :, None]

        def cbody(c, acc):
            u = u_ref[:, pl.ds(c * CSL, CSL), :]
            return acc + jnp.sum(((u & maskhi) == pref3).astype(jnp.int32),
                                 axis=1)

        acc = lax.fori_loop(0, NCH, cbody,
                            jnp.zeros((N_IMG, LN), jnp.int32))
        cnt0 = jnp.sum(acc, axis=1, keepdims=True)
        take1 = rr >= cnt0
        prefix = jnp.where(take1, prefix | bit, prefix)
        rr = jnp.where(take1, rr - cnt0, rr)
        return prefix, rr

    rank = jnp.full((N_IMG, 1), HW // 2 - 1, jnp.int32)
    u1, _ = lax.fori_loop(0, 32, bit_body, (jnp.zeros_like(rank), rank))
    s1 = (u1 ^ jnp.int32(INT_MIN))[:, :, None]

    def cbody2(c, carry):
        c_le, amin = carry
        s = u_ref[:, pl.ds(c * CSL, CSL), :] ^ jnp.int32(INT_MIN)
        c_le = c_le + jnp.sum((s <= s1).astype(jnp.int32), axis=1)
        amin = jnp.minimum(amin, jnp.min(
            jnp.where(s > s1, s, jnp.int32(INT_MAX)), axis=1))
        return c_le, amin

    c_le_l, amin_l = lax.fori_loop(
        0, NCH, cbody2,
        (jnp.zeros((N_IMG, LN), jnp.int32),
         jnp.full((N_IMG, LN), INT_MAX, jnp.int32)))
    c_le = jnp.sum(c_le_l, axis=1, keepdims=True)
    above_min = jnp.min(amin_l, axis=1, keepdims=True)
    u2 = jnp.where(c_le >= HW // 2 + 1, u1, above_min ^ jnp.int32(INT_MIN))
    med = 0.5 * (_inv_key_u(u1) + _inv_key_u(u2))
    med3 = med[:, :, None]

    def cbody3(c, acc):
        x = _inv_key_u(u_ref[:, pl.ds(c * CSL, CSL), :])
        return acc + jnp.sum(jnp.abs(x - med3), axis=1)

    sabs_l = lax.fori_loop(0, NCH, cbody3, jnp.zeros((N_IMG, LN), jnp.float32))
    sabs = jnp.sum(sabs_l, axis=1, keepdims=True)
    scale = jnp.maximum(sabs * (1.0 / HW), 1e-6)
    inv_scale = 1.0 / scale
    mp = med3[0:8]
    mt = med3[8:16]
    isp = inv_scale[0:8, :, None]
    ist = inv_scale[8:16, :, None]

    def cbody4(c, _):
        sl = pl.ds(c * CSL, CSL)
        r_ref[:, sl, :] = jnp.abs((p_ref[:, sl, :] - mp) * isp
                                  - (t_ref[:, sl, :] - mt) * ist)
        return 0

    lax.fori_loop(0, NCH, cbody4, 0)


_SC_MESH = plsc.VectorSubcoreMesh(core_axis_name="c", subcore_axis_name="s")

SC_TILES = 16
SC_EPT = N_RES // SC_TILES
SC_CHUNK = 8192
SC_NCHUNK = SC_EPT // SC_CHUNK
SC_VPC = SC_CHUNK // 16


def _sc_lane_reduce(ref1d, j, stride):
    acc = jnp.zeros((16,), jnp.int32)
    for l in range(16):
        acc = acc + ref1d[pl.ds(l * stride + j * 16, 16)]
    return acc


@functools.partial(
    pl.kernel,
    mesh=_SC_MESH,
    compiler_params=pltpu.CompilerParams(needs_layout_passes=False),
    out_type=(
        jax.ShapeDtypeStruct((2 * SC_TILES, 16), jnp.float32),
        jax.ShapeDtypeStruct((16,), jnp.int32),
    ),
    scratch_types=[
        pltpu.VMEM((16 * 2048,), jnp.int32),
        pltpu.VMEM((SC_CHUNK,), jnp.float32),
        pltpu.VMEM((2048,), jnp.int32),
        pltpu.VMEM((16 * 2048,), jnp.int32),
        pltpu.VMEM((16,), jnp.float32),
        pltpu.VMEM((16,), jnp.int32),
        pltpu.VMEM_SHARED((16 * 2048,), jnp.int32),
    ],
)
def _sc_trim(r_hbm, zeros_hbm, outf_hbm, outb_hbm,
             hist, data, mytot, tot, stage_f, stage_b, sh_tot):
    c = lax.axis_index("c")
    s = lax.axis_index("s")

    @pl.when(c == 0)
    def _():
        li = lax.iota(jnp.int32, 16)
        ones = jnp.ones((16,), jnp.int32)
        base = s * SC_EPT

        def hist_pass(shift, nbits, prefix, pshift, rrem):
            pltpu.sync_copy(zeros_hbm, hist)

            def chunk_body(ci, _):
                pltpu.sync_copy(r_hbm.at[pl.ds(base + ci * SC_CHUNK, SC_CHUNK)],
                                data)

                def vec_body(j, _):
                    for q in range(8):
                        k = lax.bitcast_convert_type(
                            data[pl.ds(j * 128 + q * 16, 16)], jnp.int32)
                        bins = (lax.shift_right_logical(k, shift)
                                & ((1 << nbits) - 1))
                        m = lax.shift_right_logical(k, pshift) == prefix
                        plsc.addupdate_scatter(hist, [li * 2048 + bins], ones,
                                               mask=m)
                    return 0

                lax.fori_loop(0, SC_VPC // 8, vec_body, 0)
                return 0

            lax.fori_loop(0, SC_NCHUNK, chunk_body, 0)

            def red_body(j, _):
                mytot[pl.ds(j * 16, 16)] = _sc_lane_reduce(hist, j, 2048)
                return 0

            lax.fori_loop(0, (1 << nbits) // 16, red_body, 0)
            plsc.subcore_barrier()
            pltpu.sync_copy(mytot.at[pl.ds(0, 1 << nbits)],
                            sh_tot.at[pl.ds(s * 2048, 1 << nbits)])
            plsc.subcore_barrier()
            pltpu.sync_copy(sh_tot, tot)

            def scan_body(j, carry):
                found, dig, run, rr = carry
                t = _sc_lane_reduce(tot, j, 2048)
                cum = plsc.cumsum(t)
                tot_j = jnp.sum(t)
                fmask = (run + cum) > rr
                lane = jnp.sum(jnp.where(fmask, 0, 1))
                anyf = lane < 16
                c_at = jnp.sum(jnp.where(li == lane, cum, 0))
                t_at = jnp.sum(jnp.where(li == lane, t, 0))
                newly = jnp.logical_and(found == 0, anyf)
                dig = jnp.where(newly, j * 16 + lane, dig)
                rr = jnp.where(newly, rr - (run + c_at - t_at), rr)
                found = jnp.where(newly, 1, found)
                run = run + tot_j
                return found, dig, run, rr

            nj = (1 << nbits) // 16
            _, dig, _, rrem = lax.fori_loop(
                0, nj, scan_body,
                (jnp.int32(0), jnp.int32(0), jnp.int32(0), rrem))
            return dig, rrem

        rr0 = jnp.int32(N_KEEP - 1)
        b1, rr1 = hist_pass(20, 11, jnp.int32(0), 31, rr0)
        b2, rr2 = hist_pass(9, 11, b1, 20, rr1)
        p12 = lax.shift_left(b1, 11) | b2
        b3, _ = hist_pass(0, 9, p12, 9, rr2)
        t_bits = lax.shift_left(p12, 9) | b3

        def sum_chunk(ci, carry):
            cacc, sacc = carry
            pltpu.sync_copy(r_hbm.at[pl.ds(base + ci * SC_CHUNK, SC_CHUNK)],
                            data)

            def vec_body(j, cc):
                ca, sa = cc
                for q in range(8):
                    v = data[pl.ds(j * 128 + q * 16, 16)]
                    k = lax.bitcast_convert_type(v, jnp.int32)
                    m = k < t_bits
                    ca = ca + jnp.where(m, 1.0, 0.0)
                    sa = sa + jnp.where(m, v, 0.0)
                return ca, sa

            return lax.fori_loop(0, SC_VPC // 8, vec_body, (cacc, sacc))

        cacc, sacc = lax.fori_loop(
            0, SC_NCHUNK, sum_chunk,
            (jnp.zeros((16,), jnp.float32), jnp.zeros((16,), jnp.float32)))
        stage_f[...] = cacc
        pltpu.sync_copy(stage_f, outf_hbm.at[s])
        stage_f[...] = sacc
        pltpu.sync_copy(stage_f, outf_hbm.at[s + 16])

        @pl.when(s == 0)
        def _():
            stage_b[...] = jnp.full((16,), 1, jnp.int32) * t_bits
            pltpu.sync_copy(stage_b, outb_hbm)


@jax.jit
def kernel(pred_depth, target, mask):
    del mask
    p = pred_depth.reshape(8, SL, LN)
    t = target.reshape(8, SL, LN)
    r = pl.pallas_call(
        _medres_body,
        out_shape=jax.ShapeDtypeStruct((8, SL, LN), jnp.float32),
        scratch_shapes=[pltpu.VMEM((N_IMG, SL, LN), jnp.int32)],
    )(p, t)
    outf, outb = _sc_trim(r.reshape(N_RES),
                          jnp.zeros((16 * 2048,), jnp.int32))
    c_lt = jnp.sum(outf[0:16])
    s_lt = jnp.sum(outf[16:32])
    t_val = lax.bitcast_convert_type(outb[0], jnp.float32)
    total = s_lt + (N_KEEP - c_lt) * t_val
    return total * (1.0 / N_RES)


# speedup vs baseline: 2.1240x; 1.9737x over previous
import jax
import jax.numpy as jnp
from jax import lax
from jax.experimental import pallas as pl
from jax.experimental.pallas import tpu as pltpu

INT_MIN = -(2**31)
INT_MAX = 2**31 - 1

N_IMG = 16
HW = 512 * 512
N_RES = 8 * HW
N_KEEP = int(N_RES * 0.8)
LN = 128
SL = HW // LN
CSL = 256
NCH = SL // CSL


def _key_u(x):
    i = lax.bitcast_convert_type(x, jnp.int32)
    return jnp.where(i >= 0, i ^ jnp.int32(INT_MIN), ~i)


def _inv_key_u(u):
    i = jnp.where(u < 0, u ^ jnp.int32(INT_MIN), ~u)
    return lax.bitcast_convert_type(i, jnp.float32)


def _fused_body(p_ref, t_ref, out_ref, u_ref, r_ref):
    def prologue(c, _):
        sl = pl.ds(c * CSL, CSL)
        u_ref[0:8, sl, :] = _key_u(p_ref[:, sl, :])
        u_ref[8:16, sl, :] = _key_u(t_ref[:, sl, :])
        return 0

    lax.fori_loop(0, NCH, prologue, 0)

    def bit_body(it, carry):
        prefix, rr = carry
        b = 31 - it
        maskhi = lax.shift_left(jnp.int32(-1), b)
        bit = lax.shift_left(jnp.int32(1), b)
        pref3 = prefix[:, :, None]

        def cbody(c, acc):
            u = u_ref[:, pl.ds(c * CSL, CSL), :]
            return acc + jnp.sum(((u & maskhi) == pref3).astype(jnp.int32),
                                 axis=1)

        acc = lax.fori_loop(0, NCH, cbody,
                            jnp.zeros((N_IMG, LN), jnp.int32))
        cnt0 = jnp.sum(acc, axis=1, keepdims=True)
        take1 = rr >= cnt0
        prefix = jnp.where(take1, prefix | bit, prefix)
        rr = jnp.where(take1, rr - cnt0, rr)
        return prefix, rr

    rank = jnp.full((N_IMG, 1), HW // 2 - 1, jnp.int32)
    u1, _ = lax.fori_loop(0, 32, bit_body, (jnp.zeros_like(rank), rank))
    s1 = (u1 ^ jnp.int32(INT_MIN))[:, :, None]

    def cbody2(c, carry):
        c_le, amin = carry
        s = u_ref[:, pl.ds(c * CSL, CSL), :] ^ jnp.int32(INT_MIN)
        c_le = c_le + jnp.sum((s <= s1).astype(jnp.int32), axis=1)
        amin = jnp.minimum(amin, jnp.min(
            jnp.where(s > s1, s, jnp.int32(INT_MAX)), axis=1))
        return c_le, amin

    c_le_l, amin_l = lax.fori_loop(
        0, NCH, cbody2,
        (jnp.zeros((N_IMG, LN), jnp.int32),
         jnp.full((N_IMG, LN), INT_MAX, jnp.int32)))
    c_le = jnp.sum(c_le_l, axis=1, keepdims=True)
    above_min = jnp.min(amin_l, axis=1, keepdims=True)
    u2 = jnp.where(c_le >= HW // 2 + 1, u1, above_min ^ jnp.int32(INT_MIN))
    med = 0.5 * (_inv_key_u(u1) + _inv_key_u(u2))
    med3 = med[:, :, None]

    def cbody3(c, acc):
        x = _inv_key_u(u_ref[:, pl.ds(c * CSL, CSL), :])
        return acc + jnp.sum(jnp.abs(x - med3), axis=1)

    sabs_l = lax.fori_loop(0, NCH, cbody3, jnp.zeros((N_IMG, LN), jnp.float32))
    sabs = jnp.sum(sabs_l, axis=1, keepdims=True)
    scale = jnp.maximum(sabs * (1.0 / HW), 1e-6)
    inv_scale = 1.0 / scale
    mp = med3[0:8]
    mt = med3[8:16]
    isp = inv_scale[0:8, :, None]
    ist = inv_scale[8:16, :, None]

    def cbody4(c, _):
        sl = pl.ds(c * CSL, CSL)
        r_ref[:, sl, :] = jnp.abs((p_ref[:, sl, :] - mp) * isp
                                  - (t_ref[:, sl, :] - mt) * ist)
        return 0

    lax.fori_loop(0, NCH, cbody4, 0)

    def bit_body2(it, carry):
        prefix, rr = carry
        b = 31 - it
        maskhi = lax.shift_left(jnp.int32(-1), b)
        bit = lax.shift_left(jnp.int32(1), b)

        def cbody(c, acc):
            u = lax.bitcast_convert_type(r_ref[:, pl.ds(c * CSL, CSL), :],
                                         jnp.int32)
            return acc + jnp.sum(((u & maskhi) == prefix).astype(jnp.int32),
                                 axis=1)

        acc = lax.fori_loop(0, NCH, cbody, jnp.zeros((8, LN), jnp.int32))
        cnt0 = jnp.sum(acc)
        take1 = rr >= cnt0
        prefix = jnp.where(take1, prefix | bit, prefix)
        rr = jnp.where(take1, rr - cnt0, rr)
        return prefix, rr

    t_bits, _ = lax.fori_loop(0, 32, bit_body2,
                              (jnp.int32(0), jnp.int32(N_KEEP - 1)))
    t_val = lax.bitcast_convert_type(t_bits, jnp.float32)

    def cbody5(c, carry):
        c_lt, s_lt = carry
        r = r_ref[:, pl.ds(c * CSL, CSL), :]
        u = lax.bitcast_convert_type(r, jnp.int32)
        below = u < t_bits
        c_lt = c_lt + jnp.sum(below.astype(jnp.float32), axis=1)
        s_lt = s_lt + jnp.sum(jnp.where(below, r, 0.0), axis=1)
        return c_lt, s_lt

    c_lt_l, s_lt_l = lax.fori_loop(0, NCH, cbody5,
                                   (jnp.zeros((8, LN), jnp.float32),
                                    jnp.zeros((8, LN), jnp.float32)))
    c_lt = jnp.sum(c_lt_l)
    s_lt = jnp.sum(s_lt_l)
    total = s_lt + (N_KEEP - c_lt) * t_val
    out_ref[...] = jnp.full((1, 1), 1.0 / N_RES) * total


@jax.jit
def kernel(pred_depth, target, mask):
    del mask
    p = pred_depth.reshape(8, SL, LN)
    t = target.reshape(8, SL, LN)
    out = pl.pallas_call(
        _fused_body,
        out_shape=jax.ShapeDtypeStruct((1, 1), jnp.float32),
        scratch_shapes=[
            pltpu.VMEM((N_IMG, SL, LN), jnp.int32),
            pltpu.VMEM((8, SL, LN), jnp.float32),
        ],
    )(p, t)
    return out.reshape(())


# speedup vs baseline: 2.1538x; 1.0140x over previous
import jax
import jax.numpy as jnp
from jax import lax
from jax.experimental import pallas as pl
from jax.experimental.pallas import tpu as pltpu

INT_MIN = -(2**31)
INT_MAX = 2**31 - 1

N_IMG = 16
HW = 512 * 512
N_RES = 8 * HW
N_KEEP = int(N_RES * 0.8)
LN = 128
SL = HW // LN
CSL = 512
NCH = SL // CSL


def _key_u(x):
    i = lax.bitcast_convert_type(x, jnp.int32)
    return jnp.where(i >= 0, i ^ jnp.int32(INT_MIN), ~i)


def _inv_key_u(u):
    i = jnp.where(u < 0, u ^ jnp.int32(INT_MIN), ~u)
    return lax.bitcast_convert_type(i, jnp.float32)


def _fused_body(p_ref, t_ref, out_ref, u_ref, r_ref):
    def prologue(c, _):
        sl = pl.ds(c * CSL, CSL)
        u_ref[0:8, sl, :] = _key_u(p_ref[:, sl, :])
        u_ref[8:16, sl, :] = _key_u(t_ref[:, sl, :])
        return 0

    lax.fori_loop(0, NCH, prologue, 0)

    def bit_body(it, carry):
        prefix, rr = carry
        b = 31 - it
        maskhi = lax.shift_left(jnp.int32(-1), b)
        bit = lax.shift_left(jnp.int32(1), b)
        pref3 = prefix[:, :, None]

        def cbody(c, acc):
            u = u_ref[:, pl.ds(c * CSL, CSL), :]
            return acc + jnp.sum(((u & maskhi) == pref3).astype(jnp.int32),
                                 axis=1)

        acc = lax.fori_loop(0, NCH, cbody,
                            jnp.zeros((N_IMG, LN), jnp.int32))
        cnt0 = jnp.sum(acc, axis=1, keepdims=True)
        take1 = rr >= cnt0
        prefix = jnp.where(take1, prefix | bit, prefix)
        rr = jnp.where(take1, rr - cnt0, rr)
        return prefix, rr

    rank = jnp.full((N_IMG, 1), HW // 2 - 1, jnp.int32)
    u1, _ = lax.fori_loop(0, 32, bit_body, (jnp.zeros_like(rank), rank))
    s1 = (u1 ^ jnp.int32(INT_MIN))[:, :, None]

    def cbody2(c, carry):
        c_le, amin = carry
        s = u_ref[:, pl.ds(c * CSL, CSL), :] ^ jnp.int32(INT_MIN)
        c_le = c_le + jnp.sum((s <= s1).astype(jnp.int32), axis=1)
        amin = jnp.minimum(amin, jnp.min(
            jnp.where(s > s1, s, jnp.int32(INT_MAX)), axis=1))
        return c_le, amin

    c_le_l, amin_l = lax.fori_loop(
        0, NCH, cbody2,
        (jnp.zeros((N_IMG, LN), jnp.int32),
         jnp.full((N_IMG, LN), INT_MAX, jnp.int32)))
    c_le = jnp.sum(c_le_l, axis=1, keepdims=True)
    above_min = jnp.min(amin_l, axis=1, keepdims=True)
    u2 = jnp.where(c_le >= HW // 2 + 1, u1, above_min ^ jnp.int32(INT_MIN))
    med = 0.5 * (_inv_key_u(u1) + _inv_key_u(u2))
    med3 = med[:, :, None]

    def cbody3(c, acc):
        x = _inv_key_u(u_ref[:, pl.ds(c * CSL, CSL), :])
        return acc + jnp.sum(jnp.abs(x - med3), axis=1)

    sabs_l = lax.fori_loop(0, NCH, cbody3, jnp.zeros((N_IMG, LN), jnp.float32))
    sabs = jnp.sum(sabs_l, axis=1, keepdims=True)
    scale = jnp.maximum(sabs * (1.0 / HW), 1e-6)
    inv_scale = 1.0 / scale
    mp = med3[0:8]
    mt = med3[8:16]
    isp = inv_scale[0:8, :, None]
    ist = inv_scale[8:16, :, None]

    def cbody4(c, _):
        sl = pl.ds(c * CSL, CSL)
        r_ref[:, sl, :] = jnp.abs((p_ref[:, sl, :] - mp) * isp
                                  - (t_ref[:, sl, :] - mt) * ist)
        return 0

    lax.fori_loop(0, NCH, cbody4, 0)

    def bit_body2(it, carry):
        prefix, rr = carry
        b = 31 - it
        maskhi = lax.shift_left(jnp.int32(-1), b)
        bit = lax.shift_left(jnp.int32(1), b)

        def cbody(c, acc):
            u = lax.bitcast_convert_type(r_ref[:, pl.ds(c * CSL, CSL), :],
                                         jnp.int32)
            return acc + jnp.sum(((u & maskhi) == prefix).astype(jnp.int32),
                                 axis=1)

        acc = lax.fori_loop(0, NCH, cbody, jnp.zeros((8, LN), jnp.int32))
        cnt0 = jnp.sum(acc)
        take1 = rr >= cnt0
        prefix = jnp.where(take1, prefix | bit, prefix)
        rr = jnp.where(take1, rr - cnt0, rr)
        return prefix, rr

    t_bits, _ = lax.fori_loop(0, 32, bit_body2,
                              (jnp.int32(0), jnp.int32(N_KEEP - 1)))
    t_val = lax.bitcast_convert_type(t_bits, jnp.float32)

    def cbody5(c, carry):
        c_lt, s_lt = carry
        r = r_ref[:, pl.ds(c * CSL, CSL), :]
        u = lax.bitcast_convert_type(r, jnp.int32)
        below = u < t_bits
        c_lt = c_lt + jnp.sum(below.astype(jnp.float32), axis=1)
        s_lt = s_lt + jnp.sum(jnp.where(below, r, 0.0), axis=1)
        return c_lt, s_lt

    c_lt_l, s_lt_l = lax.fori_loop(0, NCH, cbody5,
                                   (jnp.zeros((8, LN), jnp.float32),
                                    jnp.zeros((8, LN), jnp.float32)))
    c_lt = jnp.sum(c_lt_l)
    s_lt = jnp.sum(s_lt_l)
    total = s_lt + (N_KEEP - c_lt) * t_val
    out_ref[...] = jnp.full((1, 1), 1.0 / N_RES) * total


@jax.jit
def kernel(pred_depth, target, mask):
    del mask
    p = pred_depth.reshape(8, SL, LN)
    t = target.reshape(8, SL, LN)
    out = pl.pallas_call(
        _fused_body,
        out_shape=jax.ShapeDtypeStruct((1, 1), jnp.float32),
        scratch_shapes=[
            pltpu.VMEM((N_IMG, SL, LN), jnp.int32),
            pltpu.VMEM((8, SL, LN), jnp.float32),
        ],
    )(p, t)
    return out.reshape(())


# speedup vs baseline: 2.1983x; 1.0207x over previous
import jax
import jax.numpy as jnp
from jax import lax
from jax.experimental import pallas as pl
from jax.experimental.pallas import tpu as pltpu

INT_MIN = -(2**31)
INT_MAX = 2**31 - 1

N_IMG = 16
HW = 512 * 512
N_RES = 8 * HW
N_KEEP = int(N_RES * 0.8)
LN = 128
SL = HW // LN
CSL = 1024
NCH = SL // CSL


def _key_u(x):
    i = lax.bitcast_convert_type(x, jnp.int32)
    return jnp.where(i >= 0, i ^ jnp.int32(INT_MIN), ~i)


def _inv_key_u(u):
    i = jnp.where(u < 0, u ^ jnp.int32(INT_MIN), ~u)
    return lax.bitcast_convert_type(i, jnp.float32)


def _fused_body(p_ref, t_ref, out_ref, u_ref, r_ref):
    def prologue(c, _):
        sl = pl.ds(c * CSL, CSL)
        u_ref[0:8, sl, :] = _key_u(p_ref[:, sl, :])
        u_ref[8:16, sl, :] = _key_u(t_ref[:, sl, :])
        return 0

    lax.fori_loop(0, NCH, prologue, 0)

    def bit_body(it, carry):
        prefix, rr = carry
        b = 31 - it
        maskhi = lax.shift_left(jnp.int32(-1), b)
        bit = lax.shift_left(jnp.int32(1), b)
        pref3 = prefix[:, :, None]

        def cbody(c, acc):
            u = u_ref[:, pl.ds(c * CSL, CSL), :]
            return acc + jnp.sum(((u & maskhi) == pref3).astype(jnp.int32),
                                 axis=1)

        acc = lax.fori_loop(0, NCH, cbody,
                            jnp.zeros((N_IMG, LN), jnp.int32))
        cnt0 = jnp.sum(acc, axis=1, keepdims=True)
        take1 = rr >= cnt0
        prefix = jnp.where(take1, prefix | bit, prefix)
        rr = jnp.where(take1, rr - cnt0, rr)
        return prefix, rr

    rank = jnp.full((N_IMG, 1), HW // 2 - 1, jnp.int32)
    u1, _ = lax.fori_loop(0, 32, bit_body, (jnp.zeros_like(rank), rank))
    s1 = (u1 ^ jnp.int32(INT_MIN))[:, :, None]

    def cbody2(c, carry):
        c_le, amin = carry
        s = u_ref[:, pl.ds(c * CSL, CSL), :] ^ jnp.int32(INT_MIN)
        c_le = c_le + jnp.sum((s <= s1).astype(jnp.int32), axis=1)
        amin = jnp.minimum(amin, jnp.min(
            jnp.where(s > s1, s, jnp.int32(INT_MAX)), axis=1))
        return c_le, amin

    c_le_l, amin_l = lax.fori_loop(
        0, NCH, cbody2,
        (jnp.zeros((N_IMG, LN), jnp.int32),
         jnp.full((N_IMG, LN), INT_MAX, jnp.int32)))
    c_le = jnp.sum(c_le_l, axis=1, keepdims=True)
    above_min = jnp.min(amin_l, axis=1, keepdims=True)
    u2 = jnp.where(c_le >= HW // 2 + 1, u1, above_min ^ jnp.int32(INT_MIN))
    med = 0.5 * (_inv_key_u(u1) + _inv_key_u(u2))
    med3 = med[:, :, None]

    def cbody3(c, acc):
        x = _inv_key_u(u_ref[:, pl.ds(c * CSL, CSL), :])
        return acc + jnp.sum(jnp.abs(x - med3), axis=1)

    sabs_l = lax.fori_loop(0, NCH, cbody3, jnp.zeros((N_IMG, LN), jnp.float32))
    sabs = jnp.sum(sabs_l, axis=1, keepdims=True)
    scale = jnp.maximum(sabs * (1.0 / HW), 1e-6)
    inv_scale = 1.0 / scale
    mp = med3[0:8]
    mt = med3[8:16]
    isp = inv_scale[0:8, :, None]
    ist = inv_scale[8:16, :, None]

    def cbody4(c, _):
        sl = pl.ds(c * CSL, CSL)
        r_ref[:, sl, :] = jnp.abs((p_ref[:, sl, :] - mp) * isp
                                  - (t_ref[:, sl, :] - mt) * ist)
        return 0

    lax.fori_loop(0, NCH, cbody4, 0)

    def bit_body2(it, carry):
        prefix, rr = carry
        b = 31 - it
        maskhi = lax.shift_left(jnp.int32(-1), b)
        bit = lax.shift_left(jnp.int32(1), b)

        def cbody(c, acc):
            u = lax.bitcast_convert_type(r_ref[:, pl.ds(c * CSL, CSL), :],
                                         jnp.int32)
            return acc + jnp.sum(((u & maskhi) == prefix).astype(jnp.int32),
                                 axis=1)

        acc = lax.fori_loop(0, NCH, cbody, jnp.zeros((8, LN), jnp.int32))
        cnt0 = jnp.sum(acc)
        take1 = rr >= cnt0
        prefix = jnp.where(take1, prefix | bit, prefix)
        rr = jnp.where(take1, rr - cnt0, rr)
        return prefix, rr

    t_bits, _ = lax.fori_loop(0, 32, bit_body2,
                              (jnp.int32(0), jnp.int32(N_KEEP - 1)))
    t_val = lax.bitcast_convert_type(t_bits, jnp.float32)

    def cbody5(c, carry):
        c_lt, s_lt = carry
        r = r_ref[:, pl.ds(c * CSL, CSL), :]
        u = lax.bitcast_convert_type(r, jnp.int32)
        below = u < t_bits
        c_lt = c_lt + jnp.sum(below.astype(jnp.float32), axis=1)
        s_lt = s_lt + jnp.sum(jnp.where(below, r, 0.0), axis=1)
        return c_lt, s_lt

    c_lt_l, s_lt_l = lax.fori_loop(0, NCH, cbody5,
                                   (jnp.zeros((8, LN), jnp.float32),
                                    jnp.zeros((8, LN), jnp.float32)))
    c_lt = jnp.sum(c_lt_l)
    s_lt = jnp.sum(s_lt_l)
    total = s_lt + (N_KEEP - c_lt) * t_val
    out_ref[...] = jnp.full((1, 1), 1.0 / N_RES) * total


@jax.jit
def kernel(pred_depth, target, mask):
    del mask
    p = pred_depth.reshape(8, SL, LN)
    t = target.reshape(8, SL, LN)
    out = pl.pallas_call(
        _fused_body,
        out_shape=jax.ShapeDtypeStruct((1, 1), jnp.float32),
        scratch_shapes=[
            pltpu.VMEM((N_IMG, SL, LN), jnp.int32),
            pltpu.VMEM((8, SL, LN), jnp.float32),
        ],
    )(p, t)
    return out.reshape(())
